# baseline (speedup 1.0000x reference)
import functools

import jax
import jax.numpy as jnp
from jax import lax
from jax.experimental import pallas as pl
from jax.experimental.pallas import tpu as pltpu
from jax.experimental.pallas import tpu_sc as plsc

N = 10000
E = 320000
D = 128
H = 64
G = 128

NC = 2
NS = 16
NW = NC * NS
LANES = 16
CH = 80
NCHUNK = (E // NW) // CH
ROWS_PER_COPY = 1000
ZROWS = 125

EROWS = E // 128
CCH = EROWS // NW
CTAIL = EROWS - NW * CCH

NB = 400
NGRID = N // NB

_mesh = plsc.VectorSubcoreMesh(
    core_axis_name="c", subcore_axis_name="s", num_cores=NC, num_subcores=NS
)
_sc_params = pltpu.CompilerParams(use_tc_tiling_on_sc=False)


@functools.partial(
    pl.kernel,
    out_type=jax.ShapeDtypeStruct((NC, N, LANES), jnp.float32),
    mesh=_mesh,
    scratch_types=[
        pltpu.VMEM((NCHUNK, CH), jnp.int32),
        pltpu.VMEM((NCHUNK, CH), jnp.float32),
        pltpu.VMEM((CH, LANES), jnp.float32),
        pltpu.VMEM((ZROWS, LANES), jnp.float32),
        pltpu.VMEM_SHARED((N, LANES), jnp.float32),
        pltpu.SemaphoreType.DMA,
    ],
    compiler_params=_sc_params,
)
def _sc_deg(dst_hbm, ew_hbm, out_hbm, dst_v, ew_v, stage_v, zero_v, acc_sh, sem):
    cid = lax.axis_index("c")
    sid = lax.axis_index("s")
    wid = cid * NS + sid

    @pl.loop(0, ZROWS)
    def _(r):
        zero_v[r, :] = jnp.zeros((LANES,), jnp.float32)

    @pl.when(sid < 10)
    def _():
        for k in range(8):
            pltpu.sync_copy(
                zero_v, acc_sh.at[pl.ds(sid * ROWS_PER_COPY + k * ZROWS, ZROWS)]
            )

    plsc.subcore_barrier()

    pltpu.sync_copy(dst_hbm.at[wid], dst_v)
    pltpu.sync_copy(ew_hbm.at[wid], ew_v)

    @pl.loop(0, NCHUNK)
    def _(j):
        @pl.loop(0, CH // LANES)
        def _(g):
            wv = ew_v[j, pl.ds(g * LANES, LANES)]
            for k in range(LANES):
                stage_v[g * LANES + k, :] = jnp.full((LANES,), wv[k],
                                                     jnp.float32)

        pltpu.sync_copy(stage_v, acc_sh.at[dst_v.at[j]], add=True)

    plsc.subcore_barrier()

    @pl.when(sid < 10)
    def _():
        base = sid * ROWS_PER_COPY
        pltpu.sync_copy(
            acc_sh.at[pl.ds(base, ROWS_PER_COPY)],
            out_hbm.at[cid, pl.ds(base, ROWS_PER_COPY)],
        )


@functools.partial(
    pl.kernel,
    out_type=jax.ShapeDtypeStruct((NC, N, H), jnp.float32),
    mesh=_mesh,
    scratch_types=[
        pltpu.VMEM((CCH, 128), jnp.int32),
        pltpu.VMEM((CCH, 128), jnp.int32),
        pltpu.VMEM((CCH, 128), jnp.float32),
        pltpu.VMEM((1, 128), jnp.int32),
        pltpu.VMEM((1, 128), jnp.int32),
        pltpu.VMEM((1, 128), jnp.float32),
        pltpu.VMEM((128, H), jnp.float32),
        pltpu.VMEM((128, H), jnp.float32),
        pltpu.VMEM((ZROWS, H), jnp.float32),
        pltpu.VMEM_SHARED((N, H), jnp.float32),
        pltpu.SemaphoreType.DMA,
        pltpu.SemaphoreType.DMA,
        pltpu.SemaphoreType.DMA,
        pltpu.SemaphoreType.DMA,
    ],
    compiler_params=_sc_params,
)
def _sc_conv(y_hbm, src_hbm, dst_hbm, ew_hbm, out_hbm,
             src_v, dst_v, ew_v, srcx_v, dstx_v, ewx_v,
             rows_a, rows_b, zero_v, acc_sh,
             sem_a, sem_b, sem_sa, sem_sb):
    cid = lax.axis_index("c")
    sid = lax.axis_index("s")
    wid = cid * NS + sid

    @pl.loop(0, ZROWS)
    def _(r):
        for c in range(H // LANES):
            zero_v[r, pl.ds(c * LANES, LANES)] = jnp.zeros((LANES,), jnp.float32)

    @pl.when(sid < 10)
    def _():
        for k in range(8):
            pltpu.sync_copy(
                zero_v, acc_sh.at[pl.ds(sid * ROWS_PER_COPY + k * ZROWS, ZROWS)]
            )

    @pl.loop(0, 128)
    def _(r):
        for c in range(H // LANES):
            rows_b[r, pl.ds(c * LANES, LANES)] = jnp.zeros((LANES,), jnp.float32)

    base = wid * CCH
    pltpu.sync_copy(src_hbm.at[pl.ds(base, CCH)], src_v)
    pltpu.sync_copy(dst_hbm.at[pl.ds(base, CCH)], dst_v)
    pltpu.sync_copy(ew_hbm.at[pl.ds(base, CCH)], ew_v)

    @pl.when(wid < CTAIL)
    def _():
        xrow = NW * CCH + wid
        pltpu.sync_copy(src_hbm.at[pl.ds(xrow, 1)], srcx_v)
        pltpu.sync_copy(dst_hbm.at[pl.ds(xrow, 1)], dstx_v)
        pltpu.sync_copy(ew_hbm.at[pl.ds(xrow, 1)], ewx_v)

    plsc.subcore_barrier()

    def _scale(rows_v, ew_ref, j):
        @pl.loop(0, 128 // LANES)
        def _(g):
            wv = ew_ref[j, pl.ds(g * LANES, LANES)]
            for k in range(LANES):
                wk = jnp.full((LANES,), wv[k], jnp.float32)
                r = g * LANES + k
                for c in range(H // LANES):
                    sl = pl.ds(c * LANES, LANES)
                    rows_v[r, sl] = rows_v[r, sl] * wk

    pltpu.async_copy(rows_b, acc_sh.at[dst_v.at[0]], sem_sb, add=True)
    pltpu.async_copy(y_hbm.at[src_v.at[0]], rows_a, sem_a)

    @pl.loop(0, CCH, step=2)
    def _(j):
        pltpu.make_async_copy(y_hbm.at[src_v.at[j]], rows_a, sem_a).wait()
        pltpu.make_async_copy(rows_b, acc_sh.at[dst_v.at[j]], sem_sb).wait()
        pltpu.async_copy(y_hbm.at[src_v.at[j + 1]], rows_b, sem_b)
        _scale(rows_a, ew_v, j)
        pltpu.async_copy(rows_a, acc_sh.at[dst_v.at[j]], sem_sa, add=True)
        pltpu.make_async_copy(y_hbm.at[src_v.at[j + 1]], rows_b, sem_b).wait()
        pltpu.make_async_copy(rows_a, acc_sh.at[dst_v.at[j]], sem_sa).wait()

        @pl.when(j + 2 < CCH)
        def _():
            pltpu.async_copy(y_hbm.at[src_v.at[j + 2]], rows_a, sem_a)

        _scale(rows_b, ew_v, j + 1)
        pltpu.async_copy(rows_b, acc_sh.at[dst_v.at[j + 1]], sem_sb, add=True)

    pltpu.make_async_copy(rows_b, acc_sh.at[dst_v.at[0]], sem_sb).wait()

    @pl.when(wid < CTAIL)
    def _():
        pltpu.async_copy(y_hbm.at[srcx_v.at[0]], rows_a, sem_a).wait()
        _scale(rows_a, ewx_v, 0)
        pltpu.sync_copy(rows_a, acc_sh.at[dstx_v.at[0]], add=True)

    plsc.subcore_barrier()

    @pl.when(sid < 10)
    def _():
        base = sid * ROWS_PER_COPY
        pltpu.sync_copy(
            acc_sh.at[pl.ds(base, ROWS_PER_COPY)],
            out_hbm.at[cid, pl.ds(base, ROWS_PER_COPY)],
        )


def _ew_body(w_ref, b_ref, ea_ref, out_ref):
    a = ea_ref[...]
    w0, w1, w2 = w_ref[0, 0], w_ref[1, 0], w_ref[2, 0]
    ew = a[0] * w0 + a[1] * w1 + a[2] * w2 + b_ref[0, 0]
    out_ref[...] = jnp.maximum(ew, 0.0)


def _tc_edge_weights(ea3, emb_W, emb_b):
    return pl.pallas_call(
        _ew_body,
        out_shape=jax.ShapeDtypeStruct((E // 128, 128), jnp.float32),
        in_specs=[
            pl.BlockSpec(memory_space=pltpu.SMEM),
            pl.BlockSpec(memory_space=pltpu.SMEM),
            pl.BlockSpec((3, E // 128, 128), lambda: (0, 0, 0)),
        ],
        out_specs=pl.BlockSpec((E // 128, 128), lambda: (0, 0)),
    )(emb_W, emb_b.reshape(1, 1), ea3)


def _xw_body(x_ref, w_ref, o_ref):
    o_ref[...] = jnp.dot(x_ref[...], w_ref[...],
                         preferred_element_type=jnp.float32)


def _tc_xw(x, W1):
    return pl.pallas_call(
        _xw_body,
        grid=(NGRID,),
        out_shape=jax.ShapeDtypeStruct((N, H), jnp.float32),
        in_specs=[
            pl.BlockSpec((NB, D), lambda i: (i, 0)),
            pl.BlockSpec((D, H), lambda i: (0, 0)),
        ],
        out_specs=pl.BlockSpec((NB, H), lambda i: (i, 0)),
    )(x, W1)


def _pre_body(dp_ref, xw_ref, y_ref, dis_ref):
    deg = dp_ref[0, :, 0:1] + dp_ref[1, :, 0:1] + 1.0
    dis = lax.rsqrt(deg)
    xw = xw_ref[...]
    y_ref[...] = xw * dis
    dis_ref[...] = jnp.broadcast_to(dis, xw.shape)


def _tc_pre(degpart, xw):
    return pl.pallas_call(
        _pre_body,
        grid=(NGRID,),
        out_shape=(
            jax.ShapeDtypeStruct((N, H), jnp.float32),
            jax.ShapeDtypeStruct((N, H), jnp.float32),
        ),
        in_specs=[
            pl.BlockSpec((NC, NB, LANES), lambda i: (0, i, 0)),
            pl.BlockSpec((NB, H), lambda i: (i, 0)),
        ],
        out_specs=(
            pl.BlockSpec((NB, H), lambda i: (i, 0)),
            pl.BlockSpec((NB, H), lambda i: (i, 0)),
        ),
    )(degpart, xw)


def _mid_body(s_ref, y_ref, dis_ref, w_ref, b_ref, o_ref):
    dis = dis_ref[...]
    h = dis * (s_ref[0] + s_ref[1] + y_ref[...]) + b_ref[...]
    h = jnp.maximum(h, 0.0)
    xw2 = jnp.dot(h, w_ref[...], preferred_element_type=jnp.float32)
    o_ref[...] = dis * xw2


def _tc_mid(S1, y1, dis64, W2, b1):
    return pl.pallas_call(
        _mid_body,
        grid=(NGRID,),
        out_shape=jax.ShapeDtypeStruct((N, H), jnp.float32),
        in_specs=[
            pl.BlockSpec((NC, NB, H), lambda i: (0, i, 0)),
            pl.BlockSpec((NB, H), lambda i: (i, 0)),
            pl.BlockSpec((NB, H), lambda i: (i, 0)),
            pl.BlockSpec((H, H), lambda i: (0, 0)),
            pl.BlockSpec((1, H), lambda i: (0, 0)),
        ],
        out_specs=pl.BlockSpec((NB, H), lambda i: (i, 0)),
    )(S1, y1, dis64, W2, b1.reshape(1, H))


def _fin_body(lb_ref, s_ref, y_ref, dis_ref, b2_ref, bt_ref, lw_ref, o_ref):
    i = pl.program_id(0)
    h2 = dis_ref[...] * (s_ref[0] + s_ref[1] + y_ref[...]) + b2_ref[...]
    z = jnp.sum(h2 * lw_ref[...], axis=1, keepdims=True)
    bids = bt_ref[0]
    gids = lax.broadcasted_iota(jnp.int32, (G, 1), 0)
    oh = (bids == gids).astype(jnp.float32)
    contrib = jnp.dot(oh, z, preferred_element_type=jnp.float32)

    @pl.when(i == 0)
    def _():
        o_ref[...] = jnp.full((G, 1), lb_ref[0, 0], jnp.float32)

    o_ref[...] += contrib


def _tc_fin(S2, y2, dis64, b2, batch3, lin_W, lin_b):
    return pl.pallas_call(
        _fin_body,
        grid=(NGRID,),
        out_shape=jax.ShapeDtypeStruct((G, 1), jnp.float32),
        in_specs=[
            pl.BlockSpec(memory_space=pltpu.SMEM),
            pl.BlockSpec((NC, NB, H), lambda i: (0, i, 0)),
            pl.BlockSpec((NB, H), lambda i: (i, 0)),
            pl.BlockSpec((NB, H), lambda i: (i, 0)),
            pl.BlockSpec((1, H), lambda i: (0, 0)),
            pl.BlockSpec((1, 1, NB), lambda i: (i, 0, 0)),
            pl.BlockSpec((1, H), lambda i: (0, 0)),
        ],
        out_specs=pl.BlockSpec((G, 1), lambda i: (0, 0)),
    )(lin_b.reshape(1, 1), S2, y2, dis64, b2.reshape(1, H), batch3,
      lin_W.reshape(1, H))


def kernel(x, edge_index, edge_attr, batch, emb_W, emb_b, W1, b1, W2, b2,
           lin_W, lin_b):
    src2 = edge_index[0].reshape(EROWS, 128)
    dst2 = edge_index[1].reshape(EROWS, 128)
    ea3 = edge_attr.T.reshape(3, E // 128, 128)
    batch3 = batch.reshape(NGRID, 1, NB)

    ew2 = _tc_edge_weights(ea3, emb_W, emb_b)
    dstd = dst2.reshape(NW, NCHUNK, CH)
    ewd = ew2.reshape(NW, NCHUNK, CH)
    xw1 = _tc_xw(x, W1)
    degpart = _sc_deg(dstd, ewd)
    y1, dis64 = _tc_pre(degpart, xw1)
    S1 = _sc_conv(y1, src2, dst2, ew2)
    y2 = _tc_mid(S1, y1, dis64, W2, b1)
    S2 = _sc_conv(y2, src2, dst2, ew2)
    out = _tc_fin(S2, y2, dis64, b2, batch3, lin_W, lin_b)
    return out.reshape(G)


# speedup vs baseline: 1.2085x; 1.2085x over previous
import functools

import jax
import jax.numpy as jnp
from jax import lax
from jax.experimental import pallas as pl
from jax.experimental.pallas import tpu as pltpu
from jax.experimental.pallas import tpu_sc as plsc

N = 10000
E = 320000
D = 128
H = 64
G = 128

NC = 2
NS = 16
NW = NC * NS
LANES = 16
CH = 80
NCHUNK = (E // NW) // CH
ROWS_PER_COPY = 1000
ZROWS = 125

EROWS = E // 128
CCH = EROWS // NW
CTAIL = EROWS - NW * CCH
CPAD = CCH + 2

NB = 400
NGRID = N // NB

_mesh = plsc.VectorSubcoreMesh(
    core_axis_name="c", subcore_axis_name="s", num_cores=NC, num_subcores=NS
)
_sc_params = pltpu.CompilerParams(use_tc_tiling_on_sc=False)
_sc_conv_params = pltpu.CompilerParams(
    use_tc_tiling_on_sc=False, needs_layout_passes=False
)


@functools.partial(
    pl.kernel,
    out_type=jax.ShapeDtypeStruct((NC, N, LANES), jnp.float32),
    mesh=_mesh,
    scratch_types=[
        pltpu.VMEM((NCHUNK, CH), jnp.int32),
        pltpu.VMEM((NCHUNK, CH), jnp.float32),
        pltpu.VMEM((CH, LANES), jnp.float32),
        pltpu.VMEM((ZROWS, LANES), jnp.float32),
        pltpu.VMEM_SHARED((N, LANES), jnp.float32),
        pltpu.SemaphoreType.DMA,
    ],
    compiler_params=_sc_params,
)
def _sc_deg(dst_hbm, ew_hbm, out_hbm, dst_v, ew_v, stage_v, zero_v, acc_sh, sem):
    cid = lax.axis_index("c")
    sid = lax.axis_index("s")
    wid = cid * NS + sid

    @pl.loop(0, ZROWS)
    def _(r):
        zero_v[r, :] = jnp.zeros((LANES,), jnp.float32)

    @pl.when(sid < 10)
    def _():
        for k in range(8):
            pltpu.sync_copy(
                zero_v, acc_sh.at[pl.ds(sid * ROWS_PER_COPY + k * ZROWS, ZROWS)]
            )

    plsc.subcore_barrier()

    pltpu.sync_copy(dst_hbm.at[wid], dst_v)
    pltpu.sync_copy(ew_hbm.at[wid], ew_v)

    @pl.loop(0, NCHUNK)
    def _(j):
        @pl.loop(0, CH // LANES)
        def _(g):
            wv = ew_v[j, pl.ds(g * LANES, LANES)]
            for k in range(LANES):
                stage_v[g * LANES + k, :] = jnp.full((LANES,), wv[k],
                                                     jnp.float32)

        pltpu.sync_copy(stage_v, acc_sh.at[dst_v.at[j]], add=True)

    plsc.subcore_barrier()

    @pl.when(sid < 10)
    def _():
        base = sid * ROWS_PER_COPY
        pltpu.sync_copy(
            acc_sh.at[pl.ds(base, ROWS_PER_COPY)],
            out_hbm.at[cid, pl.ds(base, ROWS_PER_COPY)],
        )


@functools.partial(
    pl.kernel,
    out_type=jax.ShapeDtypeStruct((NC, N, H), jnp.float32),
    mesh=_mesh,
    scratch_types=[
        pltpu.VMEM((CCH, 128), jnp.int32),
        pltpu.VMEM((CCH, 128), jnp.int32),
        pltpu.VMEM((CCH, 128), jnp.float32),
        pltpu.VMEM((1, 128), jnp.int32),
        pltpu.VMEM((1, 128), jnp.int32),
        pltpu.VMEM((1, 128), jnp.float32),
        pltpu.VMEM((CPAD, 128), jnp.int32),
        pltpu.VMEM((CPAD, 128), jnp.int32),
        pltpu.VMEM((CPAD, 128), jnp.float32),
        pltpu.VMEM((128, H), jnp.float32),
        pltpu.VMEM((128, H), jnp.float32),
        pltpu.VMEM((ZROWS, H), jnp.float32),
        pltpu.VMEM_SHARED((N, H), jnp.float32),
        pltpu.SemaphoreType.DMA,
        pltpu.SemaphoreType.DMA,
        pltpu.SemaphoreType.DMA,
        pltpu.SemaphoreType.DMA,
    ],
    compiler_params=_sc_conv_params,
)
def _sc_conv(y_hbm, src_hbm, dst_hbm, ew_hbm, out_hbm,
             src_v, dst_v, ew_v, srcx_v, dstx_v, ewx_v,
             csrc_v, cdst_v, cew_v, rows_a, rows_b, zero_v, acc_sh,
             sem_a, sem_b, sem_sa, sem_sb):
    cid = lax.axis_index("c")
    sid = lax.axis_index("s")
    wid = cid * NS + sid

    z16f = jnp.zeros((LANES,), jnp.float32)
    z16i = jnp.zeros((LANES,), jnp.int32)

    @pl.loop(0, ZROWS)
    def _(r):
        for c in range(H // LANES):
            zero_v[r, pl.ds(c * LANES, LANES)] = z16f

    @pl.when(sid < 10)
    def _():
        for k in range(8):
            pltpu.sync_copy(
                zero_v, acc_sh.at[pl.ds(sid * ROWS_PER_COPY + k * ZROWS, ZROWS)]
            )

    @pl.loop(0, 128)
    def _(r):
        for c in range(H // LANES):
            rows_b[r, pl.ds(c * LANES, LANES)] = z16f

    @pl.loop(0, CPAD)
    def _(r):
        for g in range(128 // LANES):
            sl = pl.ds(g * LANES, LANES)
            csrc_v[r, sl] = z16i
            cdst_v[r, sl] = z16i
            cew_v[r, sl] = z16f

    for g in range(128 // LANES):
        sl = pl.ds(g * LANES, LANES)
        srcx_v[0, sl] = z16i
        dstx_v[0, sl] = z16i
        ewx_v[0, sl] = z16f

    base = wid * CCH
    pltpu.sync_copy(src_hbm.at[pl.ds(base, CCH)], src_v)
    pltpu.sync_copy(dst_hbm.at[pl.ds(base, CCH)], dst_v)
    pltpu.sync_copy(ew_hbm.at[pl.ds(base, CCH)], ew_v)

    @pl.when(wid < CTAIL)
    def _():
        xrow = NW * CCH + wid
        pltpu.sync_copy(src_hbm.at[pl.ds(xrow, 1)], srcx_v)
        pltpu.sync_copy(dst_hbm.at[pl.ds(xrow, 1)], dstx_v)
        pltpu.sync_copy(ew_hbm.at[pl.ds(xrow, 1)], ewx_v)

    plsc.subcore_barrier()

    def _cgroup(off16, s16, d16, w16):
        m = w16 > 0.0
        inc = plsc.cumsum(m.astype(jnp.int32))
        pos = off16 + inc - 1
        row = lax.shift_right_logical(pos, 7)
        col = lax.bitwise_and(pos, 127)
        plsc.store_scatter(csrc_v, [row, col], s16, mask=m)
        plsc.store_scatter(cdst_v, [row, col], d16, mask=m)
        plsc.store_scatter(cew_v, [row, col], w16, mask=m)
        return off16 + plsc.all_reduce_population_count(m)

    def _crow(off16, s_ref, d_ref, w_ref, r):
        for g in range(128 // LANES):
            sl = pl.ds(g * LANES, LANES)
            off16 = _cgroup(off16, s_ref[r, sl], d_ref[r, sl], w_ref[r, sl])
        return off16

    off16 = lax.fori_loop(
        0, CCH,
        lambda r, o: _crow(o, src_v, dst_v, ew_v, r),
        jnp.zeros((LANES,), jnp.int32),
    )
    off16 = _crow(off16, srcx_v, dstx_v, ewx_v, 0)

    cnt = off16[0]
    nch = lax.shift_right_logical(cnt + 127, 7)
    nch_e = nch + lax.bitwise_and(nch, 1)
    nch_e = jnp.maximum(nch_e, 2)

    def _scale(rows_v, j):
        @pl.loop(0, 128 // LANES)
        def _(g):
            wv = cew_v[j, pl.ds(g * LANES, LANES)]
            for k in range(LANES):
                wk = jnp.full((LANES,), wv[k], jnp.float32)
                r = g * LANES + k
                for c in range(H // LANES):
                    sl = pl.ds(c * LANES, LANES)
                    rows_v[r, sl] = rows_v[r, sl] * wk

    pltpu.async_copy(rows_b, acc_sh.at[cdst_v.at[0]], sem_sb, add=True)
    pltpu.async_copy(y_hbm.at[csrc_v.at[0]], rows_a, sem_a)

    def _pair(i, carry):
        j = 2 * i
        pltpu.make_async_copy(y_hbm.at[csrc_v.at[j]], rows_a, sem_a).wait()
        pltpu.make_async_copy(rows_b, acc_sh.at[cdst_v.at[j]], sem_sb).wait()
        pltpu.async_copy(y_hbm.at[csrc_v.at[j + 1]], rows_b, sem_b)
        _scale(rows_a, j)
        pltpu.async_copy(rows_a, acc_sh.at[cdst_v.at[j]], sem_sa, add=True)
        pltpu.make_async_copy(y_hbm.at[csrc_v.at[j + 1]], rows_b, sem_b).wait()
        pltpu.make_async_copy(rows_a, acc_sh.at[cdst_v.at[j]], sem_sa).wait()

        @pl.when(j + 2 < nch_e)
        def _():
            pltpu.async_copy(y_hbm.at[csrc_v.at[j + 2]], rows_a, sem_a)

        _scale(rows_b, j + 1)
        pltpu.async_copy(rows_b, acc_sh.at[cdst_v.at[j + 1]], sem_sb, add=True)
        return carry

    lax.fori_loop(0, lax.shift_right_logical(nch_e, 1), _pair, 0)

    pltpu.make_async_copy(rows_b, acc_sh.at[cdst_v.at[0]], sem_sb).wait()

    plsc.subcore_barrier()

    @pl.when(sid < 10)
    def _():
        base = sid * ROWS_PER_COPY
        pltpu.sync_copy(
            acc_sh.at[pl.ds(base, ROWS_PER_COPY)],
            out_hbm.at[cid, pl.ds(base, ROWS_PER_COPY)],
        )


def _ew_body(w_ref, b_ref, ea_ref, out_ref):
    a = ea_ref[...]
    w0, w1, w2 = w_ref[0, 0], w_ref[1, 0], w_ref[2, 0]
    ew = a[0] * w0 + a[1] * w1 + a[2] * w2 + b_ref[0, 0]
    out_ref[...] = jnp.maximum(ew, 0.0)


def _tc_edge_weights(ea3, emb_W, emb_b):
    return pl.pallas_call(
        _ew_body,
        out_shape=jax.ShapeDtypeStruct((E // 128, 128), jnp.float32),
        in_specs=[
            pl.BlockSpec(memory_space=pltpu.SMEM),
            pl.BlockSpec(memory_space=pltpu.SMEM),
            pl.BlockSpec((3, E // 128, 128), lambda: (0, 0, 0)),
        ],
        out_specs=pl.BlockSpec((E // 128, 128), lambda: (0, 0)),
    )(emb_W, emb_b.reshape(1, 1), ea3)


def _xw_body(x_ref, w_ref, o_ref):
    o_ref[...] = jnp.dot(x_ref[...], w_ref[...],
                         preferred_element_type=jnp.float32)


def _tc_xw(x, W1):
    return pl.pallas_call(
        _xw_body,
        grid=(NGRID,),
        out_shape=jax.ShapeDtypeStruct((N, H), jnp.float32),
        in_specs=[
            pl.BlockSpec((NB, D), lambda i: (i, 0)),
            pl.BlockSpec((D, H), lambda i: (0, 0)),
        ],
        out_specs=pl.BlockSpec((NB, H), lambda i: (i, 0)),
    )(x, W1)


def _pre_body(dp_ref, xw_ref, y_ref, dis_ref):
    deg = dp_ref[0, :, 0:1] + dp_ref[1, :, 0:1] + 1.0
    dis = lax.rsqrt(deg)
    xw = xw_ref[...]
    y_ref[...] = xw * dis
    dis_ref[...] = jnp.broadcast_to(dis, xw.shape)


def _tc_pre(degpart, xw):
    return pl.pallas_call(
        _pre_body,
        grid=(NGRID,),
        out_shape=(
            jax.ShapeDtypeStruct((N, H), jnp.float32),
            jax.ShapeDtypeStruct((N, H), jnp.float32),
        ),
        in_specs=[
            pl.BlockSpec((NC, NB, LANES), lambda i: (0, i, 0)),
            pl.BlockSpec((NB, H), lambda i: (i, 0)),
        ],
        out_specs=(
            pl.BlockSpec((NB, H), lambda i: (i, 0)),
            pl.BlockSpec((NB, H), lambda i: (i, 0)),
        ),
    )(degpart, xw)


def _mid_body(s_ref, y_ref, dis_ref, w_ref, b_ref, o_ref):
    dis = dis_ref[...]
    h = dis * (s_ref[0] + s_ref[1] + y_ref[...]) + b_ref[...]
    h = jnp.maximum(h, 0.0)
    xw2 = jnp.dot(h, w_ref[...], preferred_element_type=jnp.float32)
    o_ref[...] = dis * xw2


def _tc_mid(S1, y1, dis64, W2, b1):
    return pl.pallas_call(
        _mid_body,
        grid=(NGRID,),
        out_shape=jax.ShapeDtypeStruct((N, H), jnp.float32),
        in_specs=[
            pl.BlockSpec((NC, NB, H), lambda i: (0, i, 0)),
            pl.BlockSpec((NB, H), lambda i: (i, 0)),
            pl.BlockSpec((NB, H), lambda i: (i, 0)),
            pl.BlockSpec((H, H), lambda i: (0, 0)),
            pl.BlockSpec((1, H), lambda i: (0, 0)),
        ],
        out_specs=pl.BlockSpec((NB, H), lambda i: (i, 0)),
    )(S1, y1, dis64, W2, b1.reshape(1, H))


def _fin_body(lb_ref, s_ref, y_ref, dis_ref, b2_ref, bt_ref, lw_ref, o_ref):
    i = pl.program_id(0)
    h2 = dis_ref[...] * (s_ref[0] + s_ref[1] + y_ref[...]) + b2_ref[...]
    z = jnp.sum(h2 * lw_ref[...], axis=1, keepdims=True)
    bids = bt_ref[0]
    gids = lax.broadcasted_iota(jnp.int32, (G, 1), 0)
    oh = (bids == gids).astype(jnp.float32)
    contrib = jnp.dot(oh, z, preferred_element_type=jnp.float32)

    @pl.when(i == 0)
    def _():
        o_ref[...] = jnp.full((G, 1), lb_ref[0, 0], jnp.float32)

    o_ref[...] += contrib


def _tc_fin(S2, y2, dis64, b2, batch3, lin_W, lin_b):
    return pl.pallas_call(
        _fin_body,
        grid=(NGRID,),
        out_shape=jax.ShapeDtypeStruct((G, 1), jnp.float32),
        in_specs=[
            pl.BlockSpec(memory_space=pltpu.SMEM),
            pl.BlockSpec((NC, NB, H), lambda i: (0, i, 0)),
            pl.BlockSpec((NB, H), lambda i: (i, 0)),
            pl.BlockSpec((NB, H), lambda i: (i, 0)),
            pl.BlockSpec((1, H), lambda i: (0, 0)),
            pl.BlockSpec((1, 1, NB), lambda i: (i, 0, 0)),
            pl.BlockSpec((1, H), lambda i: (0, 0)),
        ],
        out_specs=pl.BlockSpec((G, 1), lambda i: (0, 0)),
    )(lin_b.reshape(1, 1), S2, y2, dis64, b2.reshape(1, H), batch3,
      lin_W.reshape(1, H))


def kernel(x, edge_index, edge_attr, batch, emb_W, emb_b, W1, b1, W2, b2,
           lin_W, lin_b):
    src2 = edge_index[0].reshape(EROWS, 128)
    dst2 = edge_index[1].reshape(EROWS, 128)
    ea3 = edge_attr.T.reshape(3, E // 128, 128)
    batch3 = batch.reshape(NGRID, 1, NB)

    ew2 = _tc_edge_weights(ea3, emb_W, emb_b)
    dstd = dst2.reshape(NW, NCHUNK, CH)
    ewd = ew2.reshape(NW, NCHUNK, CH)
    xw1 = _tc_xw(x, W1)
    degpart = _sc_deg(dstd, ewd)
    y1, dis64 = _tc_pre(degpart, xw1)
    S1 = _sc_conv(y1, src2, dst2, ew2)
    y2 = _tc_mid(S1, y1, dis64, W2, b1)
    S2 = _sc_conv(y2, src2, dst2, ew2)
    out = _tc_fin(S2, y2, dis64, b2, batch3, lin_W, lin_b)
    return out.reshape(G)
